# baseline (speedup 1.0000x reference)
import functools

import jax
import jax.numpy as jnp
from jax import lax
from jax.experimental import pallas as pl
from jax.experimental.pallas import tpu as pltpu
from jax.experimental.pallas import tpu_sc as plsc

NC = 2
NS = 16
NW = NC * NS


def _sc_gather3_slab(pat128, meas128, tim128, is_p, is_m, is_t, B):
    b_per_w = B // NW
    out_t = jax.ShapeDtypeStruct((B, 128), jnp.float32)
    mesh = plsc.VectorSubcoreMesh(core_axis_name="c", subcore_axis_name="s")

    @functools.partial(
        pl.kernel,
        mesh=mesh,
        out_type=(out_t, out_t, out_t),
        scratch_types=[
            pltpu.VMEM((b_per_w,), jnp.int32),
            pltpu.VMEM((b_per_w,), jnp.int32),
            pltpu.VMEM((b_per_w,), jnp.int32),
            pltpu.VMEM((b_per_w, 128), jnp.float32),
            pltpu.SemaphoreType.DMA,
            pltpu.SemaphoreType.DMA,
        ],
    )
    def gather_kernel(pat_hbm, meas_hbm, tim_hbm, ip_hbm, im_hbm, it_hbm,
                      pat_out, meas_out, tim_out,
                      ipv, imv, itv, buf, sem, sem_wb):
        wid = lax.axis_index("s") * NC + lax.axis_index("c")
        base = wid * b_per_w
        pltpu.sync_copy(ip_hbm.at[pl.ds(base, b_per_w)], ipv)
        pltpu.sync_copy(im_hbm.at[pl.ds(base, b_per_w)], imv)
        pltpu.sync_copy(it_hbm.at[pl.ds(base, b_per_w)], itv)

        def gather_one(tbl_hbm, idx_v, out_hbm):
            pltpu.async_copy(tbl_hbm.at[idx_v], buf, sem).wait()
            pltpu.async_copy(
                buf, out_hbm.at[pl.ds(base, b_per_w)], sem_wb).wait()

        gather_one(pat_hbm, ipv, pat_out)
        gather_one(meas_hbm, imv, meas_out)
        gather_one(tim_hbm, itv, tim_out)

    return gather_kernel(pat128, meas128, tim128, is_p, is_m, is_t)


def _extract(slab_ref, oh_ref):
    R = slab_ref.shape[0]
    x = slab_ref[...].reshape(R, 4, 4, 32)
    oh = oh_ref[...]
    val = (x[:, :, 0, :] * oh[:, :, 0:1]
           + x[:, :, 1, :] * oh[:, :, 1:2]
           + x[:, :, 2, :] * oh[:, :, 2:3]
           + x[:, :, 3, :] * oh[:, :, 3:4])
    return val.reshape(R, 128)


def _tc_combine_body(pat_ref, meas_ref, tim_ref, ohp_ref, ohm_ref, oht_ref,
                     cu_ref, cw_ref, bu_ref, bw_ref, s_ref, out_ref):
    pat4 = _extract(pat_ref, ohp_ref)
    meas4 = _extract(meas_ref, ohm_ref)
    tim4 = _extract(tim_ref, oht_ref)
    u = jnp.dot(cu_ref[...], bu_ref[...],
                preferred_element_type=jnp.float32,
                precision=lax.Precision.HIGHEST)
    w = jnp.dot(cw_ref[...], bw_ref[...],
                preferred_element_type=jnp.float32,
                precision=lax.Precision.HIGHEST)
    prod = (pat4 + u) * meas4 * (tim4 + w)
    out_ref[...] = jnp.dot(prod, s_ref[...],
                           preferred_element_type=jnp.float32,
                           precision=lax.Precision.HIGHEST)


def kernel(idx_pat, idx_meas, idx_t, cov_u, cov_w, pat_lat, meas_lat,
           time_lat, beta_u, beta_w):
    B = idx_pat.shape[0]
    D = pat_lat.shape[1]
    NU = cov_u.shape[1]
    NWc = cov_w.shape[1]
    PACK = 128 // D
    R = B // PACK

    ip = idx_pat.astype(jnp.int32)
    im = idx_meas.astype(jnp.int32)
    it = idx_t.astype(jnp.int32)

    pat128 = pat_lat.reshape(pat_lat.shape[0] // 4, 4 * D)
    meas128 = meas_lat.reshape(meas_lat.shape[0] // 4, 4 * D)
    tim128 = time_lat.reshape(time_lat.shape[0] // 4, 4 * D)

    pat_gs, meas_gs, tim_gs = _sc_gather3_slab(
        pat128, meas128, tim128, ip >> 2, im >> 2, it >> 2, B)

    patS4 = pat_gs.reshape(R, 4 * 128)
    measS4 = meas_gs.reshape(R, 4 * 128)
    timS4 = tim_gs.reshape(R, 4 * 128)
    oh_p = jax.nn.one_hot(ip & 3, 4, dtype=jnp.float32).reshape(R, 4, 4)
    oh_m = jax.nn.one_hot(im & 3, 4, dtype=jnp.float32).reshape(R, 4, 4)
    oh_t = jax.nn.one_hot(it & 3, 4, dtype=jnp.float32).reshape(R, 4, 4)
    cu4 = cov_u.reshape(R, PACK * NU)
    cw4 = cov_w.reshape(R, PACK * NWc)

    eye = jnp.eye(PACK, dtype=jnp.float32)
    bu_bd = jnp.kron(eye, beta_u)
    bw_bd = jnp.kron(eye, beta_w)
    s_mat = jnp.kron(eye, jnp.ones((D, 1), jnp.float32))

    GRID = 8
    RB = R // GRID
    row2 = lambda width: pl.BlockSpec((RB, width), lambda i: (i, 0))
    row3 = pl.BlockSpec((RB, 4, 4), lambda i: (i, 0, 0))
    full = lambda a: pl.BlockSpec(a.shape, lambda i: (0,) * a.ndim)

    out4 = pl.pallas_call(
        _tc_combine_body,
        grid=(GRID,),
        in_specs=[row2(512), row2(512), row2(512),
                  row3, row3, row3,
                  row2(PACK * NU), row2(PACK * NWc),
                  full(bu_bd), full(bw_bd), full(s_mat)],
        out_specs=pl.BlockSpec((RB, PACK), lambda i: (i, 0)),
        out_shape=jax.ShapeDtypeStruct((R, PACK), jnp.float32),
    )(patS4, measS4, timS4, oh_p, oh_m, oh_t, cu4, cw4, bu_bd, bw_bd, s_mat)
    return out4.reshape(B)


# speedup vs baseline: 2.0223x; 2.0223x over previous
import functools

import jax
import jax.numpy as jnp
from jax import lax
from jax.experimental import pallas as pl
from jax.experimental.pallas import tpu as pltpu
from jax.experimental.pallas import tpu_sc as plsc

NC = 2
NS = 16
NW = NC * NS


def _sc_gather3(pat_lat, meas_lat, time_lat, idx_pat, idx_meas, idx_t):
    B = idx_pat.shape[0]
    D = pat_lat.shape[1]
    b_per_w = B // NW
    out_t = jax.ShapeDtypeStruct((B, D), jnp.float32)
    mesh = plsc.VectorSubcoreMesh(core_axis_name="c", subcore_axis_name="s")

    @functools.partial(
        pl.kernel,
        mesh=mesh,
        out_type=(out_t, out_t, out_t),
        scratch_types=[
            pltpu.VMEM((b_per_w,), jnp.int32),
            pltpu.VMEM((b_per_w,), jnp.int32),
            pltpu.VMEM((b_per_w,), jnp.int32),
            pltpu.VMEM((b_per_w, 32), jnp.float32),
            pltpu.SemaphoreType.DMA,
            pltpu.SemaphoreType.DMA,
        ],
    )
    def gather_kernel(pat_hbm, meas_hbm, tim_hbm, ip_hbm, im_hbm, it_hbm,
                      pat_out, meas_out, tim_out,
                      ipv, imv, itv, buf, sem, sem_wb):
        wid = lax.axis_index("s") * NC + lax.axis_index("c")
        base = wid * b_per_w
        pltpu.sync_copy(ip_hbm.at[pl.ds(base, b_per_w)], ipv)
        pltpu.sync_copy(im_hbm.at[pl.ds(base, b_per_w)], imv)
        pltpu.sync_copy(it_hbm.at[pl.ds(base, b_per_w)], itv)

        def gather_one(tbl_hbm, idx_v, out_hbm):
            @pl.loop(0, b_per_w // 16)
            def _(g):
                k0 = g * 16
                iv = idx_v[pl.ds(k0, 16)]
                for j in range(16):
                    pltpu.async_copy(tbl_hbm.at[iv[j]], buf.at[k0 + j], sem)

            pltpu.make_async_copy(
                tbl_hbm.at[pl.ds(0, b_per_w)], buf, sem).wait()
            pltpu.async_copy(
                buf, out_hbm.at[pl.ds(base, b_per_w)], sem_wb).wait()

        gather_one(pat_hbm, ipv, pat_out)
        gather_one(meas_hbm, imv, meas_out)
        gather_one(tim_hbm, itv, tim_out)

    return gather_kernel(pat_lat, meas_lat, time_lat, idx_pat, idx_meas, idx_t)


def _tc_combine_body(pat_ref, meas_ref, tim_ref, cu_ref, cw_ref,
                     bu_ref, bw_ref, s_ref, out_ref):
    u = jnp.dot(cu_ref[...], bu_ref[...],
                preferred_element_type=jnp.float32,
                precision=lax.Precision.HIGHEST)
    w = jnp.dot(cw_ref[...], bw_ref[...],
                preferred_element_type=jnp.float32,
                precision=lax.Precision.HIGHEST)
    prod = (pat_ref[...] + u) * meas_ref[...] * (tim_ref[...] + w)
    out_ref[...] = jnp.dot(prod, s_ref[...],
                           preferred_element_type=jnp.float32,
                           precision=lax.Precision.HIGHEST)


def kernel(idx_pat, idx_meas, idx_t, cov_u, cov_w, pat_lat, meas_lat,
           time_lat, beta_u, beta_w):
    B = idx_pat.shape[0]
    D = pat_lat.shape[1]
    NU = cov_u.shape[1]
    NWc = cov_w.shape[1]
    PACK = 128 // D
    R = B // PACK

    pat_g, meas_g, tim_g = _sc_gather3(
        pat_lat, meas_lat, time_lat,
        idx_pat.astype(jnp.int32), idx_meas.astype(jnp.int32),
        idx_t.astype(jnp.int32))

    pat4 = pat_g.reshape(R, PACK * D)
    meas4 = meas_g.reshape(R, PACK * D)
    tim4 = tim_g.reshape(R, PACK * D)
    cu4 = cov_u.reshape(R, PACK * NU)
    cw4 = cov_w.reshape(R, PACK * NWc)

    eye = jnp.eye(PACK, dtype=jnp.float32)
    bu_bd = jnp.kron(eye, beta_u)
    bw_bd = jnp.kron(eye, beta_w)
    s_mat = jnp.kron(eye, jnp.ones((D, 1), jnp.float32))

    out4 = pl.pallas_call(
        _tc_combine_body,
        out_shape=jax.ShapeDtypeStruct((R, PACK), jnp.float32),
    )(pat4, meas4, tim4, cu4, cw4, bu_bd, bw_bd, s_mat)
    return out4.reshape(B)
